# Initial kernel scaffold; baseline (speedup 1.0000x reference)
import jax
import jax.numpy as jnp
from jax.experimental import pallas as pl


def kernel(x, key_buf, prompts_buf, num_selections, new_prompts):
    raise NotImplementedError("write your pallas kernel here")


# speedup vs baseline: 2.7450x; 2.7450x over previous
import jax
import jax.numpy as jnp
from jax.experimental import pallas as pl

POOL = 128
SEL = 5
PLEN = 5
DIM = 768
ITERS = 10
NPTS = 1025
NPAD = 1032
NQ = 1024
PD = PLEN * DIM
GROWS = NQ * SEL
GCHUNK = 256


def _mm_bt(a, b):
    return jax.lax.dot_general(a, b, (((1,), (1,)), ((), ())),
                               preferred_element_type=jnp.float32)


def _mm_at(a, b):
    return jax.lax.dot_general(a, b, (((0,), (0,)), ((), ())),
                               precision=jax.lax.Precision.HIGHEST,
                               preferred_element_type=jnp.float32)


def _main_kernel(keys_ref, x_ref, pr_ref, dist_ref, idx_ref, pm_ref):
    pts = keys_ref[...]
    row_ids = jax.lax.broadcasted_iota(jnp.int32, (NPAD, 1), 0)
    valid = (row_ids < NPTS).astype(jnp.float32)
    p2 = jnp.sum(pts * pts, axis=1, keepdims=True)
    cluster_ids = jax.lax.broadcasted_iota(jnp.int32, (NPAD, POOL), 1)
    ones_col = jnp.ones((NPAD, 1), jnp.float32)
    ones_row_d = jnp.ones((1, DIM), jnp.float32)

    def assign_onehot(cents):
        c2_row = jax.lax.dot_general(
            ones_row_d, cents * cents, (((1,), (1,)), ((), ())),
            precision=jax.lax.Precision.HIGHEST,
            preferred_element_type=jnp.float32)
        d = p2 - 2.0 * _mm_bt(pts, cents) + c2_row
        m = jnp.min(d, axis=1, keepdims=True)
        a_col = jnp.min(jnp.where(d == m, cluster_ids, POOL),
                        axis=1, keepdims=True)
        return (cluster_ids == a_col).astype(jnp.float32) * valid

    def body(_, cents):
        onehot = assign_onehot(cents)
        counts = _mm_at(onehot, ones_col)
        sums = _mm_at(onehot, pts)
        return jnp.where(counts > 0.0,
                         sums / jnp.maximum(counts, 1.0), cents)

    cents = jax.lax.fori_loop(0, ITERS, body, pts[:POOL, :])
    onehot = assign_onehot(cents)
    denom = jnp.maximum(_mm_at(onehot, ones_col), 1.0)
    key_m = _mm_at(onehot, pts) / denom
    pm_ref[...] = _mm_at(onehot, pr_ref[...]) / denom

    x = x_ref[...]
    xn = x / jnp.maximum(jnp.sqrt(jnp.sum(x * x, axis=1, keepdims=True)), 1e-8)
    kn = key_m / jnp.maximum(
        jnp.sqrt(jnp.sum(key_m * key_m, axis=1, keepdims=True)), 1e-8)
    dist = 1.0 - _mm_bt(xn, kn)
    cols = jax.lax.broadcasted_iota(jnp.int32, (NQ, POOL), 1)
    vals, idxs = [], []
    for _ in range(SEL):
        m = jnp.min(dist, axis=1, keepdims=True)
        a_col = jnp.min(jnp.where(dist == m, cols, POOL),
                        axis=1, keepdims=True)
        vals.append(m)
        idxs.append(a_col)
        dist = jnp.where(cols == a_col, jnp.float32(jnp.inf), dist)
    dist_ref[...] = jnp.concatenate(vals, axis=1)
    idx_ref[...] = jnp.concatenate(idxs, axis=1)


def _gather_kernel(idx_ref, pm_ref, out_ref):
    idx_row = idx_ref[0]
    pool_iota = jax.lax.broadcasted_iota(jnp.int32, (POOL, GCHUNK), 0)
    oh_t = (pool_iota == idx_row).astype(jnp.float32)
    out_ref[...] = _mm_at(oh_t, pm_ref[...])


def kernel(x, key_buf, prompts_buf, num_selections, new_prompts):
    key_all = jnp.concatenate([key_buf, x], axis=0)
    keys_pad = jnp.pad(key_all, ((0, NPAD - NPTS), (0, 0)))
    pr_flat = jnp.concatenate([prompts_buf.reshape(1, PD),
                               new_prompts.reshape(NQ, PD)], axis=0)
    pr_pad = jnp.pad(pr_flat, ((0, NPAD - NPTS), (0, 0)))

    dist_sel, topk, pm = pl.pallas_call(
        _main_kernel,
        out_shape=[
            jax.ShapeDtypeStruct((NQ, SEL), jnp.float32),
            jax.ShapeDtypeStruct((NQ, SEL), jnp.int32),
            jax.ShapeDtypeStruct((POOL, PD), jnp.float32),
        ],
    )(keys_pad, x, pr_pad)

    idx3 = topk.reshape(GROWS // GCHUNK, 1, GCHUNK)
    gathered = pl.pallas_call(
        _gather_kernel,
        grid=(GROWS // GCHUNK,),
        in_specs=[pl.BlockSpec((1, 1, GCHUNK), lambda i: (i, 0, 0)),
                  pl.BlockSpec((POOL, PD), lambda i: (0, 0))],
        out_specs=pl.BlockSpec((GCHUNK, PD), lambda i: (i, 0)),
        out_shape=jax.ShapeDtypeStruct((GROWS, PD), jnp.float32),
    )(idx3, pm)
    prompt = gathered.reshape(NQ, SEL, PLEN, DIM)
    return dist_sel, prompt


# speedup vs baseline: 3.6551x; 1.3316x over previous
import jax
import jax.numpy as jnp
from jax.experimental import pallas as pl

POOL = 128
SEL = 5
PLEN = 5
DIM = 768
ITERS = 10
NQ = 1024
NPTS = NQ + 1
KPAD = 8
PD = PLEN * DIM
GROWS = NQ * SEL
GCHUNK = 256


def _mm_bt(a, b):
    return jax.lax.dot_general(a, b, (((1,), (1,)), ((), ())),
                               preferred_element_type=jnp.float32)


def _mm_at(a, b):
    return jax.lax.dot_general(a, b, (((0,), (0,)), ((), ())),
                               precision=jax.lax.Precision.HIGHEST,
                               preferred_element_type=jnp.float32)


def _main_kernel(xk_ref, x_ref, pb_ref, np_ref, dist_ref, idx_ref, pm_ref):
    x = x_ref[...]
    xk = xk_ref[...]
    k_rows = jax.lax.broadcasted_iota(jnp.int32, (KPAD, 1), 0)
    k_valid = (k_rows < 1).astype(jnp.float32)
    p2x = jnp.sum(x * x, axis=1, keepdims=True)
    p2k = jnp.sum(xk * xk, axis=1, keepdims=True)
    cols_x = jax.lax.broadcasted_iota(jnp.int32, (NQ, POOL), 1)
    cols_k = jax.lax.broadcasted_iota(jnp.int32, (KPAD, POOL), 1)
    ones_x = jnp.ones((NQ, 1), jnp.float32)
    ones_row_d = jnp.ones((1, DIM), jnp.float32)

    def assign_onehots(cents):
        c2_row = jax.lax.dot_general(
            ones_row_d, cents * cents, (((1,), (1,)), ((), ())),
            precision=jax.lax.Precision.HIGHEST,
            preferred_element_type=jnp.float32)
        dx = p2x - 2.0 * _mm_bt(x, cents) + c2_row
        mx = jnp.min(dx, axis=1, keepdims=True)
        ax = jnp.min(jnp.where(dx == mx, cols_x, POOL), axis=1, keepdims=True)
        oh_x = (cols_x == ax).astype(jnp.float32)
        dk = p2k - 2.0 * _mm_bt(xk, cents) + c2_row
        mk = jnp.min(dk, axis=1, keepdims=True)
        ak = jnp.min(jnp.where(dk == mk, cols_k, POOL), axis=1, keepdims=True)
        oh_k = (cols_k == ak).astype(jnp.float32) * k_valid
        return oh_x, oh_k

    def counts_of(oh_x, oh_k):
        c = _mm_at(oh_x, ones_x) + _mm_at(oh_k, k_valid)
        return c

    def body(_, cents):
        oh_x, oh_k = assign_onehots(cents)
        counts = counts_of(oh_x, oh_k)
        sums = _mm_at(oh_k, xk) + _mm_at(oh_x, x)
        return jnp.where(counts > 0.0,
                         sums / jnp.maximum(counts, 1.0), cents)

    cents0 = jnp.concatenate([xk[:1, :], x[:POOL - 1, :]], axis=0)
    cents = jax.lax.fori_loop(0, ITERS, body, cents0)
    oh_x, oh_k = assign_onehots(cents)
    denom = jnp.maximum(counts_of(oh_x, oh_k), 1.0)
    key_m = (_mm_at(oh_k, xk) + _mm_at(oh_x, x)) / denom
    for t in range(PLEN):
        pm_ref[:, t * DIM:(t + 1) * DIM] = (
            _mm_at(oh_k, pb_ref[:, t, :]) +
            _mm_at(oh_x, np_ref[:, t, :])) / denom

    xn = x / jnp.maximum(jnp.sqrt(p2x), 1e-8)
    kn = key_m / jnp.maximum(
        jnp.sqrt(jnp.sum(key_m * key_m, axis=1, keepdims=True)), 1e-8)
    dist = 1.0 - _mm_bt(xn, kn)
    vals, idxs = [], []
    for _ in range(SEL):
        m = jnp.min(dist, axis=1, keepdims=True)
        a_col = jnp.min(jnp.where(dist == m, cols_x, POOL),
                        axis=1, keepdims=True)
        vals.append(m)
        idxs.append(a_col)
        dist = jnp.where(cols_x == a_col, jnp.float32(jnp.inf), dist)
    dist_ref[...] = jnp.concatenate(vals, axis=1)
    idx_ref[...] = jnp.concatenate(idxs, axis=1)


def _gather_kernel(idx_ref, pm_ref, out_ref):
    idx_row = idx_ref[0]
    pool_iota = jax.lax.broadcasted_iota(jnp.int32, (POOL, GCHUNK), 0)
    oh_t = (pool_iota == idx_row).astype(jnp.float32)
    for t in range(PLEN):
        out_ref[:, t, :] = _mm_at(oh_t, pm_ref[:, t * DIM:(t + 1) * DIM])


def kernel(x, key_buf, prompts_buf, num_selections, new_prompts):
    xk_pad = jnp.pad(key_buf, ((0, KPAD - 1), (0, 0)))
    pb_pad = jnp.pad(prompts_buf, ((0, KPAD - 1), (0, 0), (0, 0)))

    dist_sel, topk, pm = pl.pallas_call(
        _main_kernel,
        out_shape=[
            jax.ShapeDtypeStruct((NQ, SEL), jnp.float32),
            jax.ShapeDtypeStruct((NQ, SEL), jnp.int32),
            jax.ShapeDtypeStruct((POOL, PD), jnp.float32),
        ],
    )(xk_pad, x, pb_pad, new_prompts)

    idx3 = topk.reshape(GROWS // GCHUNK, 1, GCHUNK)
    gathered = pl.pallas_call(
        _gather_kernel,
        grid=(GROWS // GCHUNK,),
        in_specs=[pl.BlockSpec((1, 1, GCHUNK), lambda i: (i, 0, 0)),
                  pl.BlockSpec((POOL, PD), lambda i: (0, 0))],
        out_specs=pl.BlockSpec((GCHUNK, PLEN, DIM), lambda i: (i, 0, 0)),
        out_shape=jax.ShapeDtypeStruct((GROWS, PLEN, DIM), jnp.float32),
    )(idx3, pm)
    prompt = gathered.reshape(NQ, SEL, PLEN, DIM)
    return dist_sel, prompt
